# baseline (speedup 1.0000x reference)
import functools

import jax
import jax.numpy as jnp
from jax import lax
from jax.experimental import pallas as pl
from jax.experimental.pallas import tpu as pltpu
from jax.experimental.pallas import tpu_sc as plsc

N = 10000
D = 128
NC = 2
NS = 16
CK = 128
NPAD = 10240
RPT = NPAD // NS

_MESH = plsc.VectorSubcoreMesh(core_axis_name="c", subcore_axis_name="s")
_SPLIT0 = 0.324


def _agg_body(n0, n1, x_hbm, idxb, zD, agg_out,
              ib, rows, agg_sh, sem_i, sem_g, sem_s):
    c = lax.axis_index("c")
    s = lax.axis_index("s")
    nchunk = jnp.where(c == 0, n0, n1)
    pltpu.sync_copy(zD.at[pl.ds(s * RPT, RPT)], agg_sh.at[pl.ds(s * RPT, RPT)])
    plsc.subcore_barrier()

    pltpu.async_copy(idxb.at[c, s, 0], ib.at[0], sem_i.at[0]).wait()
    pltpu.async_copy(x_hbm.at[ib.at[0, 0]], rows.at[0], sem_g.at[0])

    def step(j, carry):
        b = lax.rem(j, 2)
        nb = lax.rem(j + 1, 2)

        @pl.when(j >= 1)
        def _():
            pltpu.make_async_copy(
                rows.at[nb], agg_sh.at[ib.at[nb, 1]], sem_s.at[nb]).wait()

        @pl.when(j + 1 < nchunk)
        def _():
            pltpu.async_copy(idxb.at[c, s, j + 1], ib.at[nb], sem_i.at[nb])

        pltpu.make_async_copy(
            x_hbm.at[ib.at[b, 0]], rows.at[b], sem_g.at[b]).wait()
        pltpu.async_copy(rows.at[b], agg_sh.at[ib.at[b, 1]], sem_s.at[b],
                         add=True)

        @pl.when(j + 1 < nchunk)
        def _():
            pltpu.make_async_copy(
                idxb.at[c, s, 0], ib.at[nb], sem_i.at[nb]).wait()
            pltpu.async_copy(x_hbm.at[ib.at[nb, 0]], rows.at[nb],
                             sem_g.at[nb])

        return carry

    lax.fori_loop(0, nchunk, step, 0)
    lb = lax.rem(nchunk - 1, 2)
    pltpu.make_async_copy(rows.at[lb], agg_sh.at[ib.at[lb, 1]],
                          sem_s.at[lb]).wait()
    plsc.subcore_barrier()
    pltpu.sync_copy(agg_sh.at[pl.ds(s * RPT, RPT)],
                    agg_out.at[c, pl.ds(s * RPT, RPT)])


@functools.lru_cache(maxsize=None)
def _make_agg(n0, n1):
    return pl.kernel(
        functools.partial(_agg_body, n0, n1),
        out_type=[jax.ShapeDtypeStruct((NC, NPAD, D), jnp.float32)],
        mesh=_MESH,
        scratch_types=[
            pltpu.VMEM((2, 2, CK), jnp.int32),
            pltpu.VMEM((2, CK, D), jnp.float32),
            pltpu.VMEM_SHARED((NPAD, D), jnp.float32),
            pltpu.SemaphoreType.DMA((2,)),
            pltpu.SemaphoreType.DMA((2,)),
            pltpu.SemaphoreType.DMA((2,)),
        ],
        name="sage_sc_agg",
    )


def _deg_body(nchunk, dstb, zD, o128, degw_out, didx, ones_v, degw_sh):
    c = lax.axis_index("c")
    s = lax.axis_index("s")
    pltpu.sync_copy(zD.at[pl.ds(s * RPT, RPT)],
                    degw_sh.at[pl.ds(s * RPT, RPT)])
    pltpu.sync_copy(o128, ones_v)
    pltpu.sync_copy(dstb.at[c, s], didx)
    plsc.subcore_barrier()

    def step(j, carry):
        pltpu.sync_copy(ones_v, degw_sh.at[didx.at[j]], add=True)
        return carry

    lax.fori_loop(0, nchunk, step, 0)
    plsc.subcore_barrier()
    pltpu.sync_copy(degw_sh.at[pl.ds(s * RPT, RPT)],
                    degw_out.at[c, pl.ds(s * RPT, RPT)])


@functools.lru_cache(maxsize=None)
def _make_deg(nchunk):
    return pl.kernel(
        functools.partial(_deg_body, nchunk),
        out_type=[jax.ShapeDtypeStruct((NC, NPAD, D), jnp.float32)],
        mesh=_MESH,
        scratch_types=[
            pltpu.VMEM((nchunk, CK), jnp.int32),
            pltpu.VMEM((CK, D), jnp.float32),
            pltpu.VMEM_SHARED((NPAD, D), jnp.float32),
        ],
        name="sage_sc_deg",
    )


def _dense_body(x_ref, p0_ref, p1_ref, d0_ref, d1_ref, ws_ref, wn_ref,
                b_ref, o_ref):
    deg = jnp.clip(d0_ref[...] + d1_ref[...], 1.0, None)
    hn = (p0_ref[...] + p1_ref[...]) / deg
    x = x_ref[...]
    acc = jnp.dot(x, ws_ref[...], preferred_element_type=jnp.float32)
    acc = acc + jnp.dot(hn, wn_ref[...], preferred_element_type=jnp.float32)
    o_ref[...] = jnp.maximum(acc + b_ref[...], 0.0) + x


def _dense(x, p0, p1, d0, d1, Ws, Wn, b):
    BN = 1024
    return pl.pallas_call(
        _dense_body,
        grid=(NPAD // BN,),
        in_specs=[
            pl.BlockSpec((BN, D), lambda i: (i, 0)),
            pl.BlockSpec((BN, D), lambda i: (i, 0)),
            pl.BlockSpec((BN, D), lambda i: (i, 0)),
            pl.BlockSpec((BN, 1), lambda i: (i, 0)),
            pl.BlockSpec((BN, 1), lambda i: (i, 0)),
            pl.BlockSpec((D, D), lambda i: (0, 0)),
            pl.BlockSpec((D, D), lambda i: (0, 0)),
            pl.BlockSpec((1, D), lambda i: (0, 0)),
        ],
        out_specs=pl.BlockSpec((BN, D), lambda i: (i, 0)),
        out_shape=jax.ShapeDtypeStruct((NPAD, D), jnp.float32),
        name="sage_tc_dense",
    )(x, p0, p1, d0, d1, Ws, Wn, b.reshape(1, D))


def kernel(g, feat, etype, W1s, W1n, b1, W2s, W2n, b2, W3s, W3n, b3):
    src = g[0].astype(jnp.int32)
    dst = g[1].astype(jnp.int32)
    e = src.shape[0]
    grain = NC * NS * CK
    epad = grain * -(-e // grain)
    nchunk = epad // grain
    dstb = jnp.concatenate(
        [dst, jnp.full((epad - e,), N, jnp.int32)]).reshape(NC, NS, nchunk, CK)

    T = -(-e // (NS * CK))
    n0 = max(1, min(T - 1, round(T * _SPLIT0)))
    n1 = T - n0
    nmax = max(n0, n1)
    epadT = NS * T * CK
    srcp = jnp.concatenate([src, jnp.zeros((epadT - e,), jnp.int32)])
    dstp = jnp.concatenate([dst, jnp.full((epadT - e,), N, jnp.int32)])
    idx_all = jnp.stack(
        [srcp.reshape(NS, T, CK), dstp.reshape(NS, T, CK)], axis=2)
    pad0 = jnp.zeros((NS, nmax - n0, 2, CK), jnp.int32).at[:, :, 1].set(N)
    pad1 = jnp.zeros((NS, nmax - n1, 2, CK), jnp.int32).at[:, :, 1].set(N)
    idxb = jnp.stack(
        [jnp.concatenate([idx_all[:, :n0], pad0], axis=1),
         jnp.concatenate([idx_all[:, n0:], pad1], axis=1)], axis=0)
    x0 = jnp.pad(feat, ((0, NPAD - N), (0, 0)))
    zD = jnp.zeros((NPAD, D), jnp.float32)

    agg = _make_agg(n0, n1)

    (degw,) = _make_deg(nchunk)(dstb, zD, jnp.ones((CK, D), jnp.float32))
    d0, d1 = degw[0][:, 0:1], degw[1][:, 0:1]
    (p,) = agg(x0, idxb, zD)
    h1 = _dense(x0, p[0], p[1], d0, d1, W1s, W1n, b1)
    (p,) = agg(h1, idxb, zD)
    h2 = _dense(h1, p[0], p[1], d0, d1, W2s, W2n, b2)
    (p,) = agg(h2, idxb, zD)
    h3 = _dense(h2, p[0], p[1], d0, d1, W3s, W3n, b3)
    return h3[:N]


# speedup vs baseline: 1.1913x; 1.1913x over previous
import functools

import jax
import jax.numpy as jnp
from jax import lax
from jax.experimental import pallas as pl
from jax.experimental.pallas import tpu as pltpu
from jax.experimental.pallas import tpu_sc as plsc

N = 10000
D = 128
NC = 2
NS = 16
CK = 128
NPAD = 10240
RPT = NPAD // NS

_MESH = plsc.VectorSubcoreMesh(core_axis_name="c", subcore_axis_name="s")
_SPLIT0 = 0.71


def _agg_body(n0, n1, x_hbm, idxb, zD, agg_out,
              ib, rows, agg_sh, sem_i, sem_g, sem_s):
    c = lax.axis_index("c")
    s = lax.axis_index("s")
    nchunk = jnp.where(c == 0, n0, n1)
    pltpu.sync_copy(zD.at[pl.ds(s * RPT, RPT)], agg_sh.at[pl.ds(s * RPT, RPT)])
    plsc.subcore_barrier()

    pltpu.async_copy(idxb.at[c, s, 0], ib.at[0], sem_i.at[0]).wait()
    pltpu.async_copy(x_hbm.at[ib.at[0, 0]], rows.at[0], sem_g.at[0])

    def step(j, carry):
        b = lax.rem(j, 2)
        nb = lax.rem(j + 1, 2)

        @pl.when(j >= 1)
        def _():
            pltpu.make_async_copy(
                rows.at[nb], agg_sh.at[ib.at[nb, 1]], sem_s.at[nb]).wait()

        @pl.when(j + 1 < nchunk)
        def _():
            pltpu.async_copy(idxb.at[c, s, j + 1], ib.at[nb], sem_i.at[nb])

        pltpu.make_async_copy(
            x_hbm.at[ib.at[b, 0]], rows.at[b], sem_g.at[b]).wait()
        pltpu.async_copy(rows.at[b], agg_sh.at[ib.at[b, 1]], sem_s.at[b],
                         add=True)

        @pl.when(j + 1 < nchunk)
        def _():
            pltpu.make_async_copy(
                idxb.at[c, s, 0], ib.at[nb], sem_i.at[nb]).wait()
            pltpu.async_copy(x_hbm.at[ib.at[nb, 0]], rows.at[nb],
                             sem_g.at[nb])

        return carry

    lax.fori_loop(0, nchunk, step, 0)
    lb = lax.rem(nchunk - 1, 2)
    pltpu.make_async_copy(rows.at[lb], agg_sh.at[ib.at[lb, 1]],
                          sem_s.at[lb]).wait()
    plsc.subcore_barrier()
    pltpu.sync_copy(agg_sh.at[pl.ds(s * RPT, RPT)],
                    agg_out.at[c, pl.ds(s * RPT, RPT)])


@functools.lru_cache(maxsize=None)
def _make_agg(n0, n1):
    return pl.kernel(
        functools.partial(_agg_body, n0, n1),
        out_type=[jax.ShapeDtypeStruct((NC, NPAD, D), jnp.float32)],
        mesh=_MESH,
        scratch_types=[
            pltpu.VMEM((2, 2, CK), jnp.int32),
            pltpu.VMEM((2, CK, D), jnp.float32),
            pltpu.VMEM_SHARED((NPAD, D), jnp.float32),
            pltpu.SemaphoreType.DMA((2,)),
            pltpu.SemaphoreType.DMA((2,)),
            pltpu.SemaphoreType.DMA((2,)),
        ],
        name="sage_sc_agg",
    )


def _deg_body(nchunk, dstb, zD, o128, degw_out, didx, ones_v, degw_sh):
    c = lax.axis_index("c")
    s = lax.axis_index("s")
    pltpu.sync_copy(zD.at[pl.ds(s * RPT, RPT)],
                    degw_sh.at[pl.ds(s * RPT, RPT)])
    pltpu.sync_copy(o128, ones_v)
    pltpu.sync_copy(dstb.at[c, s], didx)
    plsc.subcore_barrier()

    def step(j, carry):
        pltpu.sync_copy(ones_v, degw_sh.at[didx.at[j]], add=True)
        return carry

    lax.fori_loop(0, nchunk, step, 0)
    plsc.subcore_barrier()
    pltpu.sync_copy(degw_sh.at[pl.ds(s * RPT, RPT)],
                    degw_out.at[c, pl.ds(s * RPT, RPT)])


@functools.lru_cache(maxsize=None)
def _make_deg(nchunk):
    return pl.kernel(
        functools.partial(_deg_body, nchunk),
        out_type=[jax.ShapeDtypeStruct((NC, NPAD, D), jnp.float32)],
        mesh=_MESH,
        scratch_types=[
            pltpu.VMEM((nchunk, CK), jnp.int32),
            pltpu.VMEM((CK, D), jnp.float32),
            pltpu.VMEM_SHARED((NPAD, D), jnp.float32),
        ],
        name="sage_sc_deg",
    )


def _dense_body(x_ref, p0_ref, p1_ref, d0_ref, d1_ref, ws_ref, wn_ref,
                b_ref, o_ref):
    deg = jnp.clip(d0_ref[...] + d1_ref[...], 1.0, None)
    hn = (p0_ref[...] + p1_ref[...]) / deg
    x = x_ref[...]
    acc = jnp.dot(x, ws_ref[...], preferred_element_type=jnp.float32)
    acc = acc + jnp.dot(hn, wn_ref[...], preferred_element_type=jnp.float32)
    o_ref[...] = jnp.maximum(acc + b_ref[...], 0.0) + x


def _dense(x, p0, p1, d0, d1, Ws, Wn, b):
    BN = 1024
    return pl.pallas_call(
        _dense_body,
        grid=(NPAD // BN,),
        in_specs=[
            pl.BlockSpec((BN, D), lambda i: (i, 0)),
            pl.BlockSpec((BN, D), lambda i: (i, 0)),
            pl.BlockSpec((BN, D), lambda i: (i, 0)),
            pl.BlockSpec((BN, 1), lambda i: (i, 0)),
            pl.BlockSpec((BN, 1), lambda i: (i, 0)),
            pl.BlockSpec((D, D), lambda i: (0, 0)),
            pl.BlockSpec((D, D), lambda i: (0, 0)),
            pl.BlockSpec((1, D), lambda i: (0, 0)),
        ],
        out_specs=pl.BlockSpec((BN, D), lambda i: (i, 0)),
        out_shape=jax.ShapeDtypeStruct((NPAD, D), jnp.float32),
        name="sage_tc_dense",
    )(x, p0, p1, d0, d1, Ws, Wn, b.reshape(1, D))


def kernel(g, feat, etype, W1s, W1n, b1, W2s, W2n, b2, W3s, W3n, b3):
    src = g[0].astype(jnp.int32)
    dst = g[1].astype(jnp.int32)
    e = src.shape[0]
    grain = NC * NS * CK
    epad = grain * -(-e // grain)
    nchunk = epad // grain
    dstb = jnp.concatenate(
        [dst, jnp.full((epad - e,), N, jnp.int32)]).reshape(NC, NS, nchunk, CK)

    T = -(-e // (NS * CK))
    n0 = max(1, min(T - 1, round(T * _SPLIT0)))
    n1 = T - n0
    nmax = max(n0, n1)
    epadT = NS * T * CK
    srcp = jnp.concatenate([src, jnp.zeros((epadT - e,), jnp.int32)])
    dstp = jnp.concatenate([dst, jnp.full((epadT - e,), N, jnp.int32)])
    idx_all = jnp.stack(
        [srcp.reshape(NS, T, CK), dstp.reshape(NS, T, CK)], axis=2)
    pad0 = jnp.zeros((NS, nmax - n0, 2, CK), jnp.int32).at[:, :, 1].set(N)
    pad1 = jnp.zeros((NS, nmax - n1, 2, CK), jnp.int32).at[:, :, 1].set(N)
    idxb = jnp.stack(
        [jnp.concatenate([idx_all[:, :n0], pad0], axis=1),
         jnp.concatenate([idx_all[:, n0:], pad1], axis=1)], axis=0)
    x0 = jnp.pad(feat, ((0, NPAD - N), (0, 0)))
    zD = jnp.zeros((NPAD, D), jnp.float32)

    agg = _make_agg(n0, n1)

    (degw,) = _make_deg(nchunk)(dstb, zD, jnp.ones((CK, D), jnp.float32))
    d0, d1 = degw[0][:, 0:1], degw[1][:, 0:1]
    (p,) = agg(x0, idxb, zD)
    h1 = _dense(x0, p[0], p[1], d0, d1, W1s, W1n, b1)
    (p,) = agg(h1, idxb, zD)
    h2 = _dense(h1, p[0], p[1], d0, d1, W2s, W2n, b2)
    (p,) = agg(h2, idxb, zD)
    h3 = _dense(h2, p[0], p[1], d0, d1, W3s, W3n, b3)
    return h3[:N]


# speedup vs baseline: 1.2690x; 1.0652x over previous
import functools

import jax
import jax.numpy as jnp
from jax import lax
from jax.experimental import pallas as pl
from jax.experimental.pallas import tpu as pltpu
from jax.experimental.pallas import tpu_sc as plsc

N = 10000
D = 128
NC = 2
NS = 16
CK = 128
NPAD = 10240
RPT = NPAD // NS

_MESH = plsc.VectorSubcoreMesh(core_axis_name="c", subcore_axis_name="s")
_SPLIT0 = 0.64


def _agg_body(n0, n1, x_hbm, idxb, zD, agg_out,
              ib, rows, agg_sh, sem_i, sem_g, sem_s):
    c = lax.axis_index("c")
    s = lax.axis_index("s")
    nchunk = jnp.where(c == 0, n0, n1)
    pltpu.sync_copy(zD.at[pl.ds(s * RPT, RPT)], agg_sh.at[pl.ds(s * RPT, RPT)])
    plsc.subcore_barrier()

    pltpu.async_copy(idxb.at[c, s, 0], ib.at[0], sem_i.at[0]).wait()
    pltpu.async_copy(x_hbm.at[ib.at[0, 0]], rows.at[0], sem_g.at[0])

    def step(j, carry):
        b = lax.rem(j, 2)
        nb = lax.rem(j + 1, 2)

        @pl.when(j >= 1)
        def _():
            pltpu.make_async_copy(
                rows.at[nb], agg_sh.at[ib.at[nb, 1]], sem_s.at[nb]).wait()

        @pl.when(j + 1 < nchunk)
        def _():
            pltpu.async_copy(idxb.at[c, s, j + 1], ib.at[nb], sem_i.at[nb])

        pltpu.make_async_copy(
            x_hbm.at[ib.at[b, 0]], rows.at[b], sem_g.at[b]).wait()
        pltpu.async_copy(rows.at[b], agg_sh.at[ib.at[b, 1]], sem_s.at[b],
                         add=True)

        @pl.when(j + 1 < nchunk)
        def _():
            pltpu.make_async_copy(
                idxb.at[c, s, 0], ib.at[nb], sem_i.at[nb]).wait()
            pltpu.async_copy(x_hbm.at[ib.at[nb, 0]], rows.at[nb],
                             sem_g.at[nb])

        return carry

    lax.fori_loop(0, nchunk, step, 0)
    lb = lax.rem(nchunk - 1, 2)
    pltpu.make_async_copy(rows.at[lb], agg_sh.at[ib.at[lb, 1]],
                          sem_s.at[lb]).wait()
    plsc.subcore_barrier()
    pltpu.sync_copy(agg_sh.at[pl.ds(s * RPT, RPT)],
                    agg_out.at[c, pl.ds(s * RPT, RPT)])


@functools.lru_cache(maxsize=None)
def _make_agg(n0, n1):
    return pl.kernel(
        functools.partial(_agg_body, n0, n1),
        out_type=[jax.ShapeDtypeStruct((NC, NPAD, D), jnp.float32)],
        mesh=_MESH,
        scratch_types=[
            pltpu.VMEM((2, 2, CK), jnp.int32),
            pltpu.VMEM((2, CK, D), jnp.float32),
            pltpu.VMEM_SHARED((NPAD, D), jnp.float32),
            pltpu.SemaphoreType.DMA((2,)),
            pltpu.SemaphoreType.DMA((2,)),
            pltpu.SemaphoreType.DMA((2,)),
        ],
        name="sage_sc_agg",
    )


def _deg_body(nchunk, dstb, zD, o128, degw_out, didx, ones_v, degw_sh):
    c = lax.axis_index("c")
    s = lax.axis_index("s")
    pltpu.sync_copy(zD.at[pl.ds(s * RPT, RPT)],
                    degw_sh.at[pl.ds(s * RPT, RPT)])
    pltpu.sync_copy(o128, ones_v)
    pltpu.sync_copy(dstb.at[c, s], didx)
    plsc.subcore_barrier()

    def step(j, carry):
        pltpu.sync_copy(ones_v, degw_sh.at[didx.at[j]], add=True)
        return carry

    lax.fori_loop(0, nchunk, step, 0)
    plsc.subcore_barrier()
    pltpu.sync_copy(degw_sh.at[pl.ds(s * RPT, RPT)],
                    degw_out.at[c, pl.ds(s * RPT, RPT)])


@functools.lru_cache(maxsize=None)
def _make_deg(nchunk):
    return pl.kernel(
        functools.partial(_deg_body, nchunk),
        out_type=[jax.ShapeDtypeStruct((NC, NPAD, D), jnp.float32)],
        mesh=_MESH,
        scratch_types=[
            pltpu.VMEM((nchunk, CK), jnp.int32),
            pltpu.VMEM((CK, D), jnp.float32),
            pltpu.VMEM_SHARED((NPAD, D), jnp.float32),
        ],
        name="sage_sc_deg",
    )


def _dense_body(x_ref, p0_ref, p1_ref, d0_ref, d1_ref, ws_ref, wn_ref,
                b_ref, o_ref):
    deg = jnp.clip(d0_ref[...] + d1_ref[...], 1.0, None)
    hn = (p0_ref[...] + p1_ref[...]) / deg
    x = x_ref[...]
    acc = jnp.dot(x, ws_ref[...], preferred_element_type=jnp.float32)
    acc = acc + jnp.dot(hn, wn_ref[...], preferred_element_type=jnp.float32)
    o_ref[...] = jnp.maximum(acc + b_ref[...], 0.0) + x


def _dense(x, p0, p1, d0, d1, Ws, Wn, b):
    BN = 1024
    return pl.pallas_call(
        _dense_body,
        grid=(NPAD // BN,),
        in_specs=[
            pl.BlockSpec((BN, D), lambda i: (i, 0)),
            pl.BlockSpec((BN, D), lambda i: (i, 0)),
            pl.BlockSpec((BN, D), lambda i: (i, 0)),
            pl.BlockSpec((BN, 1), lambda i: (i, 0)),
            pl.BlockSpec((BN, 1), lambda i: (i, 0)),
            pl.BlockSpec((D, D), lambda i: (0, 0)),
            pl.BlockSpec((D, D), lambda i: (0, 0)),
            pl.BlockSpec((1, D), lambda i: (0, 0)),
        ],
        out_specs=pl.BlockSpec((BN, D), lambda i: (i, 0)),
        out_shape=jax.ShapeDtypeStruct((NPAD, D), jnp.float32),
        name="sage_tc_dense",
    )(x, p0, p1, d0, d1, Ws, Wn, b.reshape(1, D))


def kernel(g, feat, etype, W1s, W1n, b1, W2s, W2n, b2, W3s, W3n, b3):
    src = g[0].astype(jnp.int32)
    dst = g[1].astype(jnp.int32)
    e = src.shape[0]
    grain = NC * NS * CK
    epad = grain * -(-e // grain)
    nchunk = epad // grain
    dstb = jnp.concatenate(
        [dst, jnp.full((epad - e,), N, jnp.int32)]).reshape(NC, NS, nchunk, CK)

    T = -(-e // (NS * CK))
    n0 = max(1, min(T - 1, round(T * _SPLIT0)))
    n1 = T - n0
    nmax = max(n0, n1)
    epadT = NS * T * CK
    srcp = jnp.concatenate([src, jnp.zeros((epadT - e,), jnp.int32)])
    dstp = jnp.concatenate([dst, jnp.full((epadT - e,), N, jnp.int32)])
    idx_all = jnp.stack(
        [srcp.reshape(NS, T, CK), dstp.reshape(NS, T, CK)], axis=2)
    pad0 = jnp.zeros((NS, nmax - n0, 2, CK), jnp.int32).at[:, :, 1].set(N)
    pad1 = jnp.zeros((NS, nmax - n1, 2, CK), jnp.int32).at[:, :, 1].set(N)
    idxb = jnp.stack(
        [jnp.concatenate([idx_all[:, :n0], pad0], axis=1),
         jnp.concatenate([idx_all[:, n0:], pad1], axis=1)], axis=0)
    x0 = jnp.pad(feat, ((0, NPAD - N), (0, 0)))
    zD = jnp.zeros((NPAD, D), jnp.float32)

    agg = _make_agg(n0, n1)

    (degw,) = _make_deg(nchunk)(dstb, zD, jnp.ones((CK, D), jnp.float32))
    d0, d1 = degw[0][:, 0:1], degw[1][:, 0:1]
    (p,) = agg(x0, idxb, zD)
    h1 = _dense(x0, p[0], p[1], d0, d1, W1s, W1n, b1)
    (p,) = agg(h1, idxb, zD)
    h2 = _dense(h1, p[0], p[1], d0, d1, W2s, W2n, b2)
    (p,) = agg(h2, idxb, zD)
    h3 = _dense(h2, p[0], p[1], d0, d1, W3s, W3n, b3)
    return h3[:N]


# speedup vs baseline: 1.2708x; 1.0014x over previous
import functools

import jax
import jax.numpy as jnp
from jax import lax
from jax.experimental import pallas as pl
from jax.experimental.pallas import tpu as pltpu
from jax.experimental.pallas import tpu_sc as plsc

N = 10000
D = 128
NC = 2
NS = 16
CK = 128
NPAD = 10240
RPT = NPAD // NS

_MESH = plsc.VectorSubcoreMesh(core_axis_name="c", subcore_axis_name="s")
_SPLIT0 = 0.60


def _agg_body(n0, n1, x_hbm, idxb, zD, agg_out,
              ib, rows, agg_sh, sem_i, sem_g, sem_s):
    c = lax.axis_index("c")
    s = lax.axis_index("s")
    nchunk = jnp.where(c == 0, n0, n1)
    pltpu.sync_copy(zD.at[pl.ds(s * RPT, RPT)], agg_sh.at[pl.ds(s * RPT, RPT)])
    plsc.subcore_barrier()

    pltpu.async_copy(idxb.at[c, s, 0], ib.at[0], sem_i.at[0]).wait()
    pltpu.async_copy(x_hbm.at[ib.at[0, 0]], rows.at[0], sem_g.at[0])

    def step(j, carry):
        b = lax.rem(j, 2)
        nb = lax.rem(j + 1, 2)

        @pl.when(j >= 1)
        def _():
            pltpu.make_async_copy(
                rows.at[nb], agg_sh.at[ib.at[nb, 1]], sem_s.at[nb]).wait()

        @pl.when(j + 1 < nchunk)
        def _():
            pltpu.async_copy(idxb.at[c, s, j + 1], ib.at[nb], sem_i.at[nb])

        pltpu.make_async_copy(
            x_hbm.at[ib.at[b, 0]], rows.at[b], sem_g.at[b]).wait()
        pltpu.async_copy(rows.at[b], agg_sh.at[ib.at[b, 1]], sem_s.at[b],
                         add=True)

        @pl.when(j + 1 < nchunk)
        def _():
            pltpu.make_async_copy(
                idxb.at[c, s, 0], ib.at[nb], sem_i.at[nb]).wait()
            pltpu.async_copy(x_hbm.at[ib.at[nb, 0]], rows.at[nb],
                             sem_g.at[nb])

        return carry

    lax.fori_loop(0, nchunk, step, 0)
    lb = lax.rem(nchunk - 1, 2)
    pltpu.make_async_copy(rows.at[lb], agg_sh.at[ib.at[lb, 1]],
                          sem_s.at[lb]).wait()
    plsc.subcore_barrier()
    pltpu.sync_copy(agg_sh.at[pl.ds(s * RPT, RPT)],
                    agg_out.at[c, pl.ds(s * RPT, RPT)])


@functools.lru_cache(maxsize=None)
def _make_agg(n0, n1):
    return pl.kernel(
        functools.partial(_agg_body, n0, n1),
        out_type=[jax.ShapeDtypeStruct((NC, NPAD, D), jnp.float32)],
        mesh=_MESH,
        scratch_types=[
            pltpu.VMEM((2, 2, CK), jnp.int32),
            pltpu.VMEM((2, CK, D), jnp.float32),
            pltpu.VMEM_SHARED((NPAD, D), jnp.float32),
            pltpu.SemaphoreType.DMA((2,)),
            pltpu.SemaphoreType.DMA((2,)),
            pltpu.SemaphoreType.DMA((2,)),
        ],
        name="sage_sc_agg",
    )


def _deg_body(nchunk, dstb, zD, o128, degw_out, didx, ones_v, degw_sh):
    c = lax.axis_index("c")
    s = lax.axis_index("s")
    pltpu.sync_copy(zD.at[pl.ds(s * RPT, RPT)],
                    degw_sh.at[pl.ds(s * RPT, RPT)])
    pltpu.sync_copy(o128, ones_v)
    pltpu.sync_copy(dstb.at[c, s], didx)
    plsc.subcore_barrier()

    def step(j, carry):
        pltpu.sync_copy(ones_v, degw_sh.at[didx.at[j]], add=True)
        return carry

    lax.fori_loop(0, nchunk, step, 0)
    plsc.subcore_barrier()
    pltpu.sync_copy(degw_sh.at[pl.ds(s * RPT, RPT)],
                    degw_out.at[c, pl.ds(s * RPT, RPT)])


@functools.lru_cache(maxsize=None)
def _make_deg(nchunk):
    return pl.kernel(
        functools.partial(_deg_body, nchunk),
        out_type=[jax.ShapeDtypeStruct((NC, NPAD, D), jnp.float32)],
        mesh=_MESH,
        scratch_types=[
            pltpu.VMEM((nchunk, CK), jnp.int32),
            pltpu.VMEM((CK, D), jnp.float32),
            pltpu.VMEM_SHARED((NPAD, D), jnp.float32),
        ],
        name="sage_sc_deg",
    )


def _dense_body(x_ref, p0_ref, p1_ref, d0_ref, d1_ref, ws_ref, wn_ref,
                b_ref, o_ref):
    deg = jnp.clip(d0_ref[...] + d1_ref[...], 1.0, None)
    hn = (p0_ref[...] + p1_ref[...]) / deg
    x = x_ref[...]
    acc = jnp.dot(x, ws_ref[...], preferred_element_type=jnp.float32)
    acc = acc + jnp.dot(hn, wn_ref[...], preferred_element_type=jnp.float32)
    o_ref[...] = jnp.maximum(acc + b_ref[...], 0.0) + x


def _dense(x, p0, p1, d0, d1, Ws, Wn, b):
    BN = 1024
    return pl.pallas_call(
        _dense_body,
        grid=(NPAD // BN,),
        in_specs=[
            pl.BlockSpec((BN, D), lambda i: (i, 0)),
            pl.BlockSpec((BN, D), lambda i: (i, 0)),
            pl.BlockSpec((BN, D), lambda i: (i, 0)),
            pl.BlockSpec((BN, 1), lambda i: (i, 0)),
            pl.BlockSpec((BN, 1), lambda i: (i, 0)),
            pl.BlockSpec((D, D), lambda i: (0, 0)),
            pl.BlockSpec((D, D), lambda i: (0, 0)),
            pl.BlockSpec((1, D), lambda i: (0, 0)),
        ],
        out_specs=pl.BlockSpec((BN, D), lambda i: (i, 0)),
        out_shape=jax.ShapeDtypeStruct((NPAD, D), jnp.float32),
        name="sage_tc_dense",
    )(x, p0, p1, d0, d1, Ws, Wn, b.reshape(1, D))


def kernel(g, feat, etype, W1s, W1n, b1, W2s, W2n, b2, W3s, W3n, b3):
    src = g[0].astype(jnp.int32)
    dst = g[1].astype(jnp.int32)
    e = src.shape[0]
    grain = NC * NS * CK
    epad = grain * -(-e // grain)
    nchunk = epad // grain
    dstb = jnp.concatenate(
        [dst, jnp.full((epad - e,), N, jnp.int32)]).reshape(NC, NS, nchunk, CK)

    T = -(-e // (NS * CK))
    n0 = max(1, min(T - 1, round(T * _SPLIT0)))
    n1 = T - n0
    nmax = max(n0, n1)
    epadT = NS * T * CK
    srcp = jnp.concatenate([src, jnp.zeros((epadT - e,), jnp.int32)])
    dstp = jnp.concatenate([dst, jnp.full((epadT - e,), N, jnp.int32)])
    idx_all = jnp.stack(
        [srcp.reshape(NS, T, CK), dstp.reshape(NS, T, CK)], axis=2)
    pad0 = jnp.zeros((NS, nmax - n0, 2, CK), jnp.int32).at[:, :, 1].set(N)
    pad1 = jnp.zeros((NS, nmax - n1, 2, CK), jnp.int32).at[:, :, 1].set(N)
    idxb = jnp.stack(
        [jnp.concatenate([idx_all[:, :n0], pad0], axis=1),
         jnp.concatenate([idx_all[:, n0:], pad1], axis=1)], axis=0)
    x0 = jnp.pad(feat, ((0, NPAD - N), (0, 0)))
    zD = jnp.zeros((NPAD, D), jnp.float32)

    agg = _make_agg(n0, n1)

    (degw,) = _make_deg(nchunk)(dstb, zD, jnp.ones((CK, D), jnp.float32))
    d0, d1 = degw[0][:, 0:1], degw[1][:, 0:1]
    (p,) = agg(x0, idxb, zD)
    h1 = _dense(x0, p[0], p[1], d0, d1, W1s, W1n, b1)
    (p,) = agg(h1, idxb, zD)
    h2 = _dense(h1, p[0], p[1], d0, d1, W2s, W2n, b2)
    (p,) = agg(h2, idxb, zD)
    h3 = _dense(h2, p[0], p[1], d0, d1, W3s, W3n, b3)
    return h3[:N]
